# Initial kernel scaffold; baseline (speedup 1.0000x reference)
import functools

import jax
import jax.numpy as jnp
from jax import lax
from jax.experimental import pallas as pl
from jax.experimental.pallas import tpu as pltpu
from jax.experimental.pallas import tpu_sc as plsc

NC = 2
NS = 16
LANES = 16
NW = NC * NS
C = 80


def _matmul_stats_body(x_ref, w_ref, xw_ref, stats_ref):
    i = pl.program_id(0)
    xb = x_ref[...]
    r_count = w_ref.shape[0]
    for r in range(r_count):
        xw_ref[r] = jnp.dot(xb, w_ref[r], preferred_element_type=jnp.float32)
    s1 = jnp.sum(xb, axis=0, keepdims=True)
    s2 = jnp.sum(xb * xb, axis=0, keepdims=True)
    blk = jnp.concatenate([s1, s2], axis=0)

    @pl.when(i == 0)
    def _():
        stats_ref[...] = blk

    @pl.when(i > 0)
    def _():
        stats_ref[...] = stats_ref[...] + blk


def _tc_matmul_stats(x, W, bn):
    N, D = x.shape
    R = W.shape[0]
    nb = N // bn
    return pl.pallas_call(
        _matmul_stats_body,
        grid=(nb,),
        in_specs=[
            pl.BlockSpec((bn, D), lambda i: (i, 0)),
            pl.BlockSpec((R, D, D), lambda i: (0, 0, 0)),
        ],
        out_specs=[
            pl.BlockSpec((R, bn, D), lambda i: (0, i, 0)),
            pl.BlockSpec((2, D), lambda i: (0, 0)),
        ],
        out_shape=[
            jax.ShapeDtypeStruct((R, N, D), jnp.float32),
            jax.ShapeDtypeStruct((2, D), jnp.float32),
        ],
    )(x, W)


def _combine_body(n_rows, x_ref, part_ref, stats_ref, gamma_ref, beta_ref,
                  out_ref):
    st = stats_ref[...]
    inv_n = jnp.float32(1.0 / n_rows)
    mean = st[0:1] * inv_n
    var = st[1:2] * inv_n - mean * mean
    inv = lax.rsqrt(var + jnp.float32(1e-5))
    p = part_ref[0] + part_ref[1]
    h = (x_ref[...] - mean) * (inv * gamma_ref[...]) + beta_ref[...] + p
    out_ref[...] = jnp.maximum(h, jnp.float32(0.0))


def _tc_combine(x, partials, stats, gamma, beta, bn):
    N, D = x.shape
    nb = N // bn
    return pl.pallas_call(
        functools.partial(_combine_body, N),
        grid=(nb,),
        in_specs=[
            pl.BlockSpec((bn, D), lambda i: (i, 0)),
            pl.BlockSpec((2, bn, D), lambda i: (0, i, 0)),
            pl.BlockSpec((2, D), lambda i: (0, 0)),
            pl.BlockSpec((1, D), lambda i: (0, 0)),
            pl.BlockSpec((1, D), lambda i: (0, 0)),
        ],
        out_specs=pl.BlockSpec((bn, D), lambda i: (i, 0)),
        out_shape=jax.ShapeDtypeStruct((N, D), jnp.float32),
    )(x, partials, stats, gamma, beta)


def _sc_edges(table, edata, zeros, N, D, G):
    rows_per_tile = N // NS
    mesh = plsc.VectorSubcoreMesh(core_axis_name="c", subcore_axis_name="s")

    @functools.partial(
        pl.kernel,
        mesh=mesh,
        out_type=jax.ShapeDtypeStruct((NC, N, D), jnp.float32),
        scratch_types=[
            pltpu.VMEM((4, C), jnp.int32),
            pltpu.VMEM((C,), jnp.int32),
            pltpu.VMEM((C,), jnp.int32),
            pltpu.VMEM((C,), jnp.float32),
            pltpu.VMEM((C, D), jnp.float32),
            pltpu.VMEM_SHARED((N, D), jnp.float32),
            pltpu.SemaphoreType.DMA,
        ],
    )
    def k(table_hbm, edata_hbm, zeros_hbm, out_hbm,
          ebuf, idx_v, dst_v, norm_v, rows_v, acc, sem):
        cid = lax.axis_index("c")
        sid = lax.axis_index("s")
        wid = cid * NS + sid

        zbase = sid * rows_per_tile
        pltpu.sync_copy(zeros_hbm.at[pl.ds(zbase, rows_per_tile)],
                        acc.at[pl.ds(zbase, rows_per_tile)])
        plsc.subcore_barrier()

        def step(g, carry):
            pltpu.sync_copy(edata_hbm.at[wid, g], ebuf)
            for j in range(C // LANES):
                sl = pl.ds(j * LANES, LANES)
                idx_v[sl] = ebuf[1, sl] * N + ebuf[0, sl]
                dst_v[sl] = ebuf[2, sl]
                norm_v[sl] = plsc.bitcast(ebuf[3, sl], jnp.float32)
            pltpu.async_copy(table_hbm.at[idx_v], rows_v, sem).wait()

            def scale(e, c2):
                nb = jnp.full((LANES,), norm_v[e], jnp.float32)
                for v in range(D // LANES):
                    sl = pl.ds(v * LANES, LANES)
                    rows_v[e, sl] = rows_v[e, sl] * nb
                return c2

            lax.fori_loop(0, C, scale, 0)
            pltpu.sync_copy(rows_v, acc.at[dst_v], add=True)
            return carry

        lax.fori_loop(0, G, step, 0)
        plsc.subcore_barrier()
        pltpu.sync_copy(acc.at[pl.ds(zbase, rows_per_tile)],
                        out_hbm.at[cid, pl.ds(zbase, rows_per_tile)])

    return k(table, edata, zeros)


def kernel(x, edge_index, rel_type, norm, W, gamma, beta):
    N, D = x.shape
    E = rel_type.shape[0]
    R = W.shape[0]
    assert E % (NW * C) == 0 and N % NS == 0
    G = E // (NW * C)

    xw, stats = _tc_matmul_stats(x, W, bn=1000)
    table = xw.reshape(R * N, D)

    norm_bits = lax.bitcast_convert_type(norm, jnp.int32)
    edata = jnp.stack([edge_index[0], rel_type, edge_index[1], norm_bits],
                      axis=1)
    edata = edata.reshape(NW, G, C, 4).transpose(0, 1, 3, 2)

    zeros = jnp.zeros((N, D), jnp.float32)
    partials = _sc_edges(table, edata, zeros, N, D, G)

    return _tc_combine(x, partials, stats, gamma.reshape(1, D),
                       beta.reshape(1, D), bn=1000)


# speedup vs baseline: 14.5046x; 14.5046x over previous
import functools

import jax
import jax.numpy as jnp
from jax import lax
from jax.experimental import pallas as pl
from jax.experimental.pallas import tpu as pltpu
from jax.experimental.pallas import tpu_sc as plsc

NC = 2
NS = 16
LANES = 16
NW = NC * NS
C = 80


def _matmul_stats_body(x_ref, w_ref, xw_ref, stats_ref):
    i = pl.program_id(0)
    xb = x_ref[...]
    r_count = w_ref.shape[0]
    for r in range(r_count):
        xw_ref[r] = jnp.dot(xb, w_ref[r], preferred_element_type=jnp.float32)
    s1 = jnp.sum(xb, axis=0, keepdims=True)
    s2 = jnp.sum(xb * xb, axis=0, keepdims=True)
    blk = jnp.concatenate([s1, s2], axis=0)

    @pl.when(i == 0)
    def _():
        stats_ref[...] = blk

    @pl.when(i > 0)
    def _():
        stats_ref[...] = stats_ref[...] + blk


def _tc_matmul_stats(x, W, bn):
    N, D = x.shape
    R = W.shape[0]
    nb = N // bn
    return pl.pallas_call(
        _matmul_stats_body,
        grid=(nb,),
        in_specs=[
            pl.BlockSpec((bn, D), lambda i: (i, 0)),
            pl.BlockSpec((R, D, D), lambda i: (0, 0, 0)),
        ],
        out_specs=[
            pl.BlockSpec((R, bn, D), lambda i: (0, i, 0)),
            pl.BlockSpec((2, D), lambda i: (0, 0)),
        ],
        out_shape=[
            jax.ShapeDtypeStruct((R, N, D), jnp.float32),
            jax.ShapeDtypeStruct((2, D), jnp.float32),
        ],
    )(x, W)


def _combine_body(n_rows, x_ref, part_ref, stats_ref, gamma_ref, beta_ref,
                  out_ref):
    st = stats_ref[...]
    inv_n = jnp.float32(1.0 / n_rows)
    mean = st[0:1] * inv_n
    var = st[1:2] * inv_n - mean * mean
    inv = lax.rsqrt(var + jnp.float32(1e-5))
    p = part_ref[0] + part_ref[1]
    h = (x_ref[...] - mean) * (inv * gamma_ref[...]) + beta_ref[...] + p
    out_ref[...] = jnp.maximum(h, jnp.float32(0.0))


def _tc_combine(x, partials, stats, gamma, beta, bn):
    N, D = x.shape
    nb = N // bn
    return pl.pallas_call(
        functools.partial(_combine_body, N),
        grid=(nb,),
        in_specs=[
            pl.BlockSpec((bn, D), lambda i: (i, 0)),
            pl.BlockSpec((2, bn, D), lambda i: (0, i, 0)),
            pl.BlockSpec((2, D), lambda i: (0, 0)),
            pl.BlockSpec((1, D), lambda i: (0, 0)),
            pl.BlockSpec((1, D), lambda i: (0, 0)),
        ],
        out_specs=pl.BlockSpec((bn, D), lambda i: (i, 0)),
        out_shape=jax.ShapeDtypeStruct((N, D), jnp.float32),
    )(x, partials, stats, gamma, beta)


def _sc_edges(table, edata, normd, zeros, N, NPAD, D, G):
    rows_per_tile = NPAD // NS
    mesh = plsc.VectorSubcoreMesh(core_axis_name="c", subcore_axis_name="s")

    @functools.partial(
        pl.kernel,
        mesh=mesh,
        out_type=jax.ShapeDtypeStruct((NC, NPAD, D), jnp.float32),
        scratch_types=[
            pltpu.VMEM((3, C), jnp.int32),
            pltpu.VMEM((C,), jnp.int32),
            pltpu.VMEM((C,), jnp.int32),
            pltpu.VMEM((C,), jnp.float32),
            pltpu.VMEM((C, D), jnp.float32),
            pltpu.VMEM_SHARED((NPAD, D), jnp.float32),
            pltpu.SemaphoreType.DMA,
        ],
    )
    def k(table_hbm, edata_hbm, norm_hbm, zeros_hbm, out_hbm,
          ebuf, idx_v, dst_v, norm_v, rows_v, acc, sem):
        cid = lax.axis_index("c")
        sid = lax.axis_index("s")
        wid = cid * NS + sid

        zbase = sid * rows_per_tile
        pltpu.sync_copy(zeros_hbm.at[pl.ds(zbase, rows_per_tile)],
                        acc.at[pl.ds(zbase, rows_per_tile)])
        plsc.subcore_barrier()

        def step(g, carry):
            pltpu.sync_copy(edata_hbm.at[wid, g], ebuf)
            pltpu.sync_copy(norm_hbm.at[wid, g], norm_v)
            for j in range(C // LANES):
                sl = pl.ds(j * LANES, LANES)
                idx_v[sl] = ebuf[1, sl] * N + ebuf[0, sl]
                dst_v[sl] = ebuf[2, sl]
            pltpu.async_copy(table_hbm.at[idx_v], rows_v, sem).wait()

            def scale(j, c2):
                base = j * LANES
                norm16 = norm_v[pl.ds(base, LANES)]
                for t in range(LANES):
                    nb = norm16[t]
                    for v in range(D // LANES):
                        sl = pl.ds(v * LANES, LANES)
                        rows_v[base + t, sl] = rows_v[base + t, sl] * nb
                return c2

            lax.fori_loop(0, C // LANES, scale, 0)
            pltpu.sync_copy(rows_v, acc.at[dst_v], add=True)
            return carry

        lax.fori_loop(0, G, step, 0)
        plsc.subcore_barrier()
        pltpu.sync_copy(acc.at[pl.ds(zbase, rows_per_tile)],
                        out_hbm.at[cid, pl.ds(zbase, rows_per_tile)])

    return k(table, edata, normd, zeros)


def kernel(x, edge_index, rel_type, norm, W, gamma, beta):
    N, D = x.shape
    E = rel_type.shape[0]
    R = W.shape[0]
    assert E % (NW * C) == 0
    G = E // (NW * C)
    NPAD = -(-N // (NS * 8)) * (NS * 8)

    xw, stats = _tc_matmul_stats(x, W, bn=1000)
    table = xw.reshape(R * N, D)

    edata = jnp.stack([edge_index[0], rel_type, edge_index[1]], axis=1)
    edata = edata.reshape(NW, G, C, 3).transpose(0, 1, 3, 2)
    normd = norm.reshape(NW, G, C)

    zeros = jnp.zeros((NPAD, D), jnp.float32)
    partials = _sc_edges(table, edata, normd, zeros, N, NPAD, D, G)

    return _tc_combine(x, partials, stats, gamma.reshape(1, D),
                       beta.reshape(1, D), bn=1000)
